# Initial kernel scaffold; baseline (speedup 1.0000x reference)
import jax
import jax.numpy as jnp
from jax.experimental import pallas as pl


def kernel(user, item, gmf_user_emb, gmf_item_emb, mlp_user_emb, mlp_item_emb, W1, b1, W2, b2, W3, b3, Wp, bp):
    raise NotImplementedError("write your pallas kernel here")


# speedup vs baseline: 1.0624x; 1.0624x over previous
import functools

import jax
import jax.numpy as jnp
from jax import lax
from jax.experimental import pallas as pl
from jax.experimental.pallas import tpu as pltpu
from jax.experimental.pallas import tpu_sc as plsc

B = 16384
D_MLP = 128
D_GMF = 32

_info = plsc.get_sparse_core_info()
NC, NS = _info.num_cores, _info.num_subcores
NW = NC * NS
BPW = B // NW

_sc_mesh = plsc.VectorSubcoreMesh(core_axis_name="c", subcore_axis_name="s")


@functools.partial(
    pl.kernel,
    mesh=_sc_mesh,
    out_type=[
        jax.ShapeDtypeStruct((B, D_MLP), jnp.float32),
        jax.ShapeDtypeStruct((B, D_MLP), jnp.float32),
        jax.ShapeDtypeStruct((B, 128), jnp.float32),
        jax.ShapeDtypeStruct((B, 128), jnp.float32),
    ],
    scratch_types=[
        pltpu.VMEM((BPW,), jnp.int32),
        pltpu.VMEM((BPW,), jnp.int32),
        pltpu.VMEM((BPW, D_MLP), jnp.float32),
        pltpu.SemaphoreType.DMA,
    ],
)
def _sc_gather(user_hbm, item_hbm, ub_hbm, ib_hbm, mue_hbm, mie_hbm,
               gue_hbm, gie_hbm,
               mu_out, mi_out, gu_out, gi_out,
               idx_a, idx_b, buf, sem):
    wid = lax.axis_index("s") * NC + lax.axis_index("c")
    base = wid * BPW
    pltpu.sync_copy(user_hbm.at[pl.ds(base, BPW)], idx_a)
    pltpu.sync_copy(item_hbm.at[pl.ds(base, BPW)], idx_b)

    pltpu.async_copy(mue_hbm.at[idx_a], buf, sem).wait()
    pltpu.sync_copy(buf, mu_out.at[pl.ds(base, BPW)])
    pltpu.async_copy(mie_hbm.at[idx_b], buf, sem).wait()
    pltpu.sync_copy(buf, mi_out.at[pl.ds(base, BPW)])

    pltpu.sync_copy(ub_hbm.at[pl.ds(base, BPW)], idx_a)
    pltpu.sync_copy(ib_hbm.at[pl.ds(base, BPW)], idx_b)

    pltpu.async_copy(gue_hbm.at[idx_a], buf, sem).wait()
    pltpu.sync_copy(buf, gu_out.at[pl.ds(base, BPW)])
    pltpu.async_copy(gie_hbm.at[idx_b], buf, sem).wait()
    pltpu.sync_copy(buf, gi_out.at[pl.ds(base, BPW)])


BLK = 2048


def _mlp_body(mu, mi, gub, gib, uo, io, w1a, w1b, b1, w2, b2, w3, b3,
              wpg, wpx, bp, out):
    x = jnp.dot(mu[...], w1a[...], preferred_element_type=jnp.float32)
    x = x + jnp.dot(mi[...], w1b[...], preferred_element_type=jnp.float32)
    x = jnp.maximum(x + b1[...], 0.0)
    x = jnp.maximum(
        jnp.dot(x, w2[...], preferred_element_type=jnp.float32) + b2[...], 0.0)
    x = jnp.maximum(
        jnp.dot(x, w3[...], preferred_element_type=jnp.float32) + b3[...], 0.0)
    gu = jnp.zeros((BLK, D_GMF), jnp.float32)
    gi = jnp.zeros((BLK, D_GMF), jnp.float32)
    for c in range(4):
        gu = jnp.where(uo[...] == c, gub[:, c * D_GMF:(c + 1) * D_GMF], gu)
        gi = jnp.where(io[...] == c, gib[:, c * D_GMF:(c + 1) * D_GMF], gi)
    g = gu * gi
    logit = (jnp.sum(g * wpg[...], axis=1)
             + jnp.sum(x * wpx[...], axis=1) + bp[0, 0])
    out[...] = 1.0 / (1.0 + jnp.exp(-logit))


def _run_mlp(mu, mi, gub, gib, uo, io,
             w1a, w1b, b1, w2, b2, w3, b3, wpg, wpx, bp):
    grid = B // BLK
    row = lambda i: (i, 0)
    full = lambda i: (0, 0)
    return pl.pallas_call(
        _mlp_body,
        grid=(grid,),
        in_specs=[
            pl.BlockSpec((BLK, D_MLP), row),
            pl.BlockSpec((BLK, D_MLP), row),
            pl.BlockSpec((BLK, 128), row),
            pl.BlockSpec((BLK, 128), row),
            pl.BlockSpec((BLK, 1), row),
            pl.BlockSpec((BLK, 1), row),
            pl.BlockSpec((D_MLP, 128), full),
            pl.BlockSpec((D_MLP, 128), full),
            pl.BlockSpec((1, 128), full),
            pl.BlockSpec((128, 64), full),
            pl.BlockSpec((1, 64), full),
            pl.BlockSpec((64, 32), full),
            pl.BlockSpec((1, 32), full),
            pl.BlockSpec((1, 32), full),
            pl.BlockSpec((1, 32), full),
            pl.BlockSpec((1, 1), full),
        ],
        out_specs=pl.BlockSpec((BLK,), lambda i: (i,)),
        out_shape=jax.ShapeDtypeStruct((B,), jnp.float32),
    )(mu, mi, gub, gib, uo, io, w1a, w1b, b1, w2, b2, w3, b3, wpg, wpx, bp)


def kernel(user, item, gmf_user_emb, gmf_item_emb, mlp_user_emb, mlp_item_emb,
           W1, b1, W2, b2, W3, b3, Wp, bp):
    user = user.astype(jnp.int32)
    item = item.astype(jnp.int32)
    ub, uo = user >> 2, user & 3
    ib, io = item >> 2, item & 3
    gue = gmf_user_emb.reshape(25000, 128)
    gie = gmf_item_emb.reshape(25000, 128)
    mu, mi, gub, gib = _sc_gather(user, item, ub, ib,
                                  mlp_user_emb, mlp_item_emb, gue, gie)
    w1t = W1.T
    w1a, w1b = w1t[:D_MLP], w1t[D_MLP:]
    wpg = Wp[:, :D_GMF]
    wpx = Wp[:, D_GMF:]
    return _run_mlp(mu, mi, gub, gib, uo.reshape(-1, 1), io.reshape(-1, 1),
                    w1a, w1b, b1.reshape(1, -1),
                    W2.T, b2.reshape(1, -1), W3.T, b3.reshape(1, -1),
                    wpg, wpx, bp.reshape(1, 1))
